# baseline (speedup 1.0000x reference)
import functools

import jax
import jax.numpy as jnp
from jax import lax
from jax.experimental import pallas as pl
from jax.experimental.pallas import tpu as pltpu
from jax.experimental.pallas import tpu_sc as plsc

_NC = 2
_NS = 16
_NW = _NC * _NS


@functools.lru_cache(maxsize=None)
def _make_gather(G: int, C: int, D: int):
    mesh = plsc.VectorSubcoreMesh(core_axis_name="c", subcore_axis_name="s")

    @functools.partial(
        pl.kernel,
        out_type=jax.ShapeDtypeStruct((_NW, G, C, D), jnp.float32),
        mesh=mesh,
        scratch_types=[
            pltpu.VMEM((G, C), jnp.int32),
            pltpu.VMEM((2, C, D), jnp.float32),
            pltpu.SemaphoreType.DMA,
            pltpu.SemaphoreType.DMA,
        ],
        compiler_params=pltpu.CompilerParams(use_tc_tiling_on_sc=False),
    )
    def k(table_hbm, idx_hbm, out_hbm, idx_v, rows_v, gsem, ssem):
        wid = lax.axis_index("s") * _NC + lax.axis_index("c")
        pltpu.sync_copy(idx_hbm.at[wid], idx_v)

        pltpu.async_copy(table_hbm.at[idx_v.at[0]], rows_v.at[0], gsem)

        @pl.loop(0, G)
        def _(g):
            buf = lax.rem(g, 2)
            nbuf = lax.rem(g + 1, 2)

            @pl.when(g + 1 < G)
            def _():
                @pl.when(g >= 1)
                def _():
                    pltpu.make_async_copy(
                        rows_v.at[nbuf], out_hbm.at[wid, g - 1], ssem
                    ).wait()
                pltpu.async_copy(
                    table_hbm.at[idx_v.at[g + 1]], rows_v.at[nbuf], gsem
                )

            pltpu.make_async_copy(
                table_hbm.at[idx_v.at[g]], rows_v.at[buf], gsem
            ).wait()
            pltpu.async_copy(rows_v.at[buf], out_hbm.at[wid, g], ssem)

        pltpu.make_async_copy(rows_v.at[0], out_hbm.at[wid, G - 2], ssem).wait()
        pltpu.make_async_copy(rows_v.at[1], out_hbm.at[wid, G - 1], ssem).wait()

    return k


def kernel(x, W):
    B, T = x.shape
    V, D = W.shape
    n = B * T
    C = 128
    assert n % (_NW * C) == 0
    G = n // (_NW * C)
    xf = x.reshape(_NW, G, C).astype(jnp.int32)
    out = _make_gather(G, C, D)(W, xf)
    return out.reshape(B, T, D)


# speedup vs baseline: 1.0209x; 1.0209x over previous
import functools

import jax
import jax.numpy as jnp
from jax import lax
from jax.experimental import pallas as pl
from jax.experimental.pallas import tpu as pltpu
from jax.experimental.pallas import tpu_sc as plsc

_NC = 2
_NS = 16
_NW = _NC * _NS
_NBUF = 4


@functools.lru_cache(maxsize=None)
def _make_gather(G: int, C: int, D: int):
    mesh = plsc.VectorSubcoreMesh(core_axis_name="c", subcore_axis_name="s")

    @functools.partial(
        pl.kernel,
        out_type=jax.ShapeDtypeStruct((_NW, G, C, D), jnp.float32),
        mesh=mesh,
        scratch_types=[
            pltpu.VMEM((G, C), jnp.int32),
            pltpu.VMEM((_NBUF, C, D), jnp.float32),
            pltpu.SemaphoreType.DMA((_NBUF,)),
            pltpu.SemaphoreType.DMA((_NBUF,)),
        ],
        compiler_params=pltpu.CompilerParams(use_tc_tiling_on_sc=False),
    )
    def k(table_hbm, idx_hbm, out_hbm, idx_v, rows_v, gsem, ssem):
        wid = lax.axis_index("s") * _NC + lax.axis_index("c")
        pltpu.sync_copy(idx_hbm.at[wid], idx_v)

        for p in range(_NBUF - 1):
            pltpu.async_copy(table_hbm.at[idx_v.at[p]], rows_v.at[p], gsem.at[p])

        @pl.loop(0, G)
        def _(g):
            buf = lax.rem(g, _NBUF)
            nbuf = lax.rem(g + _NBUF - 1, _NBUF)

            @pl.when(g + _NBUF - 1 < G)
            def _():
                @pl.when(g >= 1)
                def _():
                    pltpu.make_async_copy(
                        rows_v.at[nbuf], out_hbm.at[wid, g - 1], ssem.at[nbuf]
                    ).wait()
                pltpu.async_copy(
                    table_hbm.at[idx_v.at[g + _NBUF - 1]],
                    rows_v.at[nbuf],
                    gsem.at[nbuf],
                )

            pltpu.make_async_copy(
                table_hbm.at[idx_v.at[g]], rows_v.at[buf], gsem.at[buf]
            ).wait()
            pltpu.async_copy(rows_v.at[buf], out_hbm.at[wid, g], ssem.at[buf])

        for p in range(_NBUF - 1):
            c = G - 1 - p
            pltpu.make_async_copy(
                rows_v.at[c % _NBUF], out_hbm.at[wid, c], ssem.at[c % _NBUF]
            ).wait()

    return k


def kernel(x, W):
    B, T = x.shape
    V, D = W.shape
    n = B * T
    C = 128
    assert n % (_NW * C) == 0
    G = n // (_NW * C)
    xf = x.reshape(_NW, G, C).astype(jnp.int32)
    out = _make_gather(G, C, D)(W, xf)
    return out.reshape(B, T, D)
